# Initial kernel scaffold; baseline (speedup 1.0000x reference)
import jax
import jax.numpy as jnp
from jax.experimental import pallas as pl


def kernel(node_specie, embeddings, atomic_masses):
    raise NotImplementedError("write your pallas kernel here")


# speedup vs baseline: 4.1840x; 4.1840x over previous
import functools

import jax
import jax.numpy as jnp
from jax import lax
from jax.experimental import pallas as pl
from jax.experimental.pallas import tpu as pltpu
from jax.experimental.pallas import tpu_sc as plsc

_N_NODES = 100000
_NUM_SPECIES = 119
_DIM = 128

_info = plsc.get_sparse_core_info()
_NC = _info.num_cores
_NS = _info.num_subcores
_NW = _NC * _NS

_CHUNK = 125
_CHUNKS_PER_W = _N_NODES // (_NW * _CHUNK)
_ROWS = _NW * _CHUNKS_PER_W


def _prescale_body(emb_ref, mass_ref, out_ref):
    out_ref[...] = emb_ref[...] * mass_ref[...] * (1.0 / 90.0)


_mesh = plsc.VectorSubcoreMesh(core_axis_name="c", subcore_axis_name="s")


@functools.partial(
    pl.kernel,
    mesh=_mesh,
    out_type=jax.ShapeDtypeStruct((_ROWS, _CHUNK, _DIM), jnp.float32),
    scratch_types=[
        pltpu.VMEM((_CHUNK,), jnp.int32),
        pltpu.VMEM((_CHUNK, _DIM), jnp.float32),
        pltpu.SemaphoreType.DMA,
    ],
)
def _sc_gather(table_hbm, idx_hbm, out_hbm, idx_v, rows_v, sem):
    wid = lax.axis_index("s") * _NC + lax.axis_index("c")

    def body(k, carry):
        chunk = wid * _CHUNKS_PER_W + k
        pltpu.sync_copy(idx_hbm.at[chunk], idx_v)
        pltpu.async_copy(table_hbm.at[idx_v], rows_v, sem).wait()
        pltpu.sync_copy(rows_v, out_hbm.at[chunk])
        return carry

    lax.fori_loop(0, _CHUNKS_PER_W, body, 0)


def kernel(node_specie, embeddings, atomic_masses):
    scaled = pl.pallas_call(
        _prescale_body,
        out_shape=jax.ShapeDtypeStruct((_NUM_SPECIES, _DIM), jnp.float32),
    )(embeddings, atomic_masses[:_NUM_SPECIES].reshape(_NUM_SPECIES, 1))
    idx2d = node_specie.reshape(_ROWS, _CHUNK)
    return _sc_gather(scaled, idx2d).reshape(_N_NODES, _DIM)


# speedup vs baseline: 4.1983x; 1.0034x over previous
import functools

import jax
import jax.numpy as jnp
from jax import lax
from jax.experimental import pallas as pl
from jax.experimental.pallas import tpu as pltpu
from jax.experimental.pallas import tpu_sc as plsc

_N_NODES = 100000
_NUM_SPECIES = 119
_DIM = 128

_info = plsc.get_sparse_core_info()
_NC = _info.num_cores
_NS = _info.num_subcores
_NW = _NC * _NS

_CHUNK = 125
_CHUNKS_PER_W = _N_NODES // (_NW * _CHUNK)
_ROWS = _NW * _CHUNKS_PER_W


def _prescale_body(emb_ref, mass_ref, out_ref):
    out_ref[...] = emb_ref[...] * mass_ref[...] * (1.0 / 90.0)


_mesh = plsc.VectorSubcoreMesh(core_axis_name="c", subcore_axis_name="s")


_NBUF = 2


@functools.partial(
    pl.kernel,
    mesh=_mesh,
    out_type=jax.ShapeDtypeStruct((_ROWS, _CHUNK, _DIM), jnp.float32),
    scratch_types=[
        pltpu.VMEM((_CHUNKS_PER_W, _CHUNK), jnp.int32),
        *[pltpu.VMEM((_CHUNK, _DIM), jnp.float32) for _ in range(_NBUF)],
        *[pltpu.SemaphoreType.DMA for _ in range(2 * _NBUF + 1)],
    ],
)
def _sc_gather(table_hbm, idx_hbm, out_hbm, idx_v, *bufs_and_sems):
    rows = bufs_and_sems[:_NBUF]
    sem_i = bufs_and_sems[_NBUF]
    sem_g = bufs_and_sems[_NBUF + 1:2 * _NBUF + 1]
    sem_o = bufs_and_sems[2 * _NBUF + 1:]
    wid = lax.axis_index("s") * _NC + lax.axis_index("c")

    pltpu.async_copy(idx_hbm.at[wid], idx_v, sem_i).wait()

    gathers = [None] * _CHUNKS_PER_W
    outs = [None] * _CHUNKS_PER_W

    def start_gather(k):
        b = k % _NBUF
        gathers[k] = pltpu.async_copy(
            table_hbm.at[idx_v.at[k]], rows[b], sem_g[b])

    start_gather(0)
    for k in range(_CHUNKS_PER_W):
        b = k % _NBUF
        gathers[k].wait()
        outs[k] = pltpu.async_copy(
            rows[b], out_hbm.at[wid * _CHUNKS_PER_W + k], sem_o[b])
        if k + 1 < _CHUNKS_PER_W:
            if k + 1 - _NBUF >= 0:
                outs[k + 1 - _NBUF].wait()
            start_gather(k + 1)
    for k in range(max(0, _CHUNKS_PER_W - _NBUF), _CHUNKS_PER_W):
        outs[k].wait()


def kernel(node_specie, embeddings, atomic_masses):
    scaled = pl.pallas_call(
        _prescale_body,
        out_shape=jax.ShapeDtypeStruct((_NUM_SPECIES, _DIM), jnp.float32),
    )(embeddings, atomic_masses[:_NUM_SPECIES].reshape(_NUM_SPECIES, 1))
    idx3d = node_specie.reshape(_NW, _CHUNKS_PER_W, _CHUNK)
    return _sc_gather(scaled, idx3d).reshape(_N_NODES, _DIM)


# speedup vs baseline: 6.8923x; 1.6417x over previous
import functools

import jax
import jax.numpy as jnp
from jax import lax
from jax.experimental import pallas as pl
from jax.experimental.pallas import tpu as pltpu
from jax.experimental.pallas import tpu_sc as plsc

_N_NODES = 100000
_NUM_SPECIES = 119
_DIM = 128

_info = plsc.get_sparse_core_info()
_NC = _info.num_cores
_NS = _info.num_subcores
_NW = _NC * _NS

_CHUNK = 128
_FULL = _N_NODES // _CHUNK
_TAIL = _N_NODES - _FULL * _CHUNK
_K_LO = _FULL // _NW
_N_HI = _FULL - _K_LO * _NW
_K_HI = _K_LO + 1


def _prescale_body(emb_ref, mass_ref, out_ref):
    out_ref[...] = emb_ref[...] * mass_ref[...] * (1.0 / 90.0)


_mesh = plsc.VectorSubcoreMesh(core_axis_name="c", subcore_axis_name="s")


_NBUF = 2


@functools.partial(
    pl.kernel,
    mesh=_mesh,
    out_type=jax.ShapeDtypeStruct((_N_NODES, _DIM), jnp.float32),
    scratch_types=[
        pltpu.VMEM((_K_HI * _CHUNK,), jnp.int32),
        pltpu.VMEM((_TAIL,), jnp.int32),
        pltpu.VMEM((_TAIL, _DIM), jnp.float32),
        *[pltpu.VMEM((_CHUNK, _DIM), jnp.float32) for _ in range(_NBUF)],
        *[pltpu.SemaphoreType.DMA for _ in range(2 * _NBUF + 2)],
    ],
)
def _sc_gather(table_hbm, idx_hbm, out_hbm, idx_v, idx_t, rows_t,
               *bufs_and_sems):
    rows = bufs_and_sems[:_NBUF]
    sem_i = bufs_and_sems[_NBUF]
    sem_t = bufs_and_sems[_NBUF + 1]
    sem_g = bufs_and_sems[_NBUF + 2:2 * _NBUF + 2]
    sem_o = bufs_and_sems[2 * _NBUF + 2:]
    wid = lax.axis_index("s") * _NC + lax.axis_index("c")

    is_hi = wid < _N_HI
    base = lax.select(is_hi, wid * _K_HI,
                      _N_HI * _K_HI + (wid - _N_HI) * _K_LO)
    row0 = base * _CHUNK

    def run(nchunks):
        pltpu.async_copy(
            idx_hbm.at[pl.ds(row0, nchunks * _CHUNK)],
            idx_v.at[pl.ds(0, nchunks * _CHUNK)], sem_i).wait()
        gathers = [None] * nchunks
        outs = [None] * nchunks

        def start_gather(k):
            b = k % _NBUF
            gathers[k] = pltpu.async_copy(
                table_hbm.at[idx_v.at[pl.ds(k * _CHUNK, _CHUNK)]],
                rows[b], sem_g[b])

        start_gather(0)
        for k in range(nchunks):
            b = k % _NBUF
            gathers[k].wait()
            outs[k] = pltpu.async_copy(
                rows[b], out_hbm.at[pl.ds(row0 + k * _CHUNK, _CHUNK)],
                sem_o[b])
            if k + 1 < nchunks:
                if k + 1 - _NBUF >= 0:
                    outs[k + 1 - _NBUF].wait()
                start_gather(k + 1)
        for k in range(max(0, nchunks - _NBUF), nchunks):
            outs[k].wait()

    @pl.when(is_hi)
    def _():
        run(_K_HI)

    @pl.when(jnp.logical_not(is_hi))
    def _():
        run(_K_LO)

    @pl.when(wid == _NW - 1)
    def _():
        t0 = _FULL * _CHUNK
        pltpu.async_copy(idx_hbm.at[pl.ds(t0, _TAIL)], idx_t, sem_t).wait()
        pltpu.async_copy(table_hbm.at[idx_t], rows_t, sem_t).wait()
        pltpu.async_copy(rows_t, out_hbm.at[pl.ds(t0, _TAIL)], sem_t).wait()


def kernel(node_specie, embeddings, atomic_masses):
    scaled = pl.pallas_call(
        _prescale_body,
        out_shape=jax.ShapeDtypeStruct((_NUM_SPECIES, _DIM), jnp.float32),
    )(embeddings, atomic_masses[:_NUM_SPECIES].reshape(_NUM_SPECIES, 1))
    return _sc_gather(scaled, node_specie)


# speedup vs baseline: 6.9066x; 1.0021x over previous
import functools

import jax
import jax.numpy as jnp
from jax import lax
from jax.experimental import pallas as pl
from jax.experimental.pallas import tpu as pltpu
from jax.experimental.pallas import tpu_sc as plsc

_N_NODES = 100000
_NUM_SPECIES = 119
_DIM = 128

_info = plsc.get_sparse_core_info()
_NC = _info.num_cores
_NS = _info.num_subcores
_NW = _NC * _NS

_CHUNK = 128
_FULL = _N_NODES // _CHUNK
_TAIL = _N_NODES - _FULL * _CHUNK
_K_LO = _FULL // _NW
_N_HI = _FULL - _K_LO * _NW
_K_HI = _K_LO + 1


def _prescale_body(emb_ref, mass_ref, out_ref):
    out_ref[...] = emb_ref[...] * mass_ref[...] * (1.0 / 90.0)


_mesh = plsc.VectorSubcoreMesh(core_axis_name="c", subcore_axis_name="s")


_NBUF = 4


@functools.partial(
    pl.kernel,
    mesh=_mesh,
    out_type=jax.ShapeDtypeStruct((_N_NODES, _DIM), jnp.float32),
    scratch_types=[
        pltpu.VMEM((_K_HI * _CHUNK,), jnp.int32),
        pltpu.VMEM((_TAIL,), jnp.int32),
        pltpu.VMEM((_TAIL, _DIM), jnp.float32),
        *[pltpu.VMEM((_CHUNK, _DIM), jnp.float32) for _ in range(_NBUF)],
        *[pltpu.SemaphoreType.DMA for _ in range(2 * _NBUF + 2)],
    ],
)
def _sc_gather(table_hbm, idx_hbm, out_hbm, idx_v, idx_t, rows_t,
               *bufs_and_sems):
    rows = bufs_and_sems[:_NBUF]
    sem_i = bufs_and_sems[_NBUF]
    sem_t = bufs_and_sems[_NBUF + 1]
    sem_g = bufs_and_sems[_NBUF + 2:2 * _NBUF + 2]
    sem_o = bufs_and_sems[2 * _NBUF + 2:]
    wid = lax.axis_index("s") * _NC + lax.axis_index("c")

    is_hi = wid < _N_HI
    base = lax.select(is_hi, wid * _K_HI,
                      _N_HI * _K_HI + (wid - _N_HI) * _K_LO)
    row0 = base * _CHUNK

    def run(nchunks):
        pltpu.async_copy(
            idx_hbm.at[pl.ds(row0, nchunks * _CHUNK)],
            idx_v.at[pl.ds(0, nchunks * _CHUNK)], sem_i).wait()
        gathers = [None] * nchunks
        outs = [None] * nchunks

        def start_gather(k):
            b = k % _NBUF
            gathers[k] = pltpu.async_copy(
                table_hbm.at[idx_v.at[pl.ds(k * _CHUNK, _CHUNK)]],
                rows[b], sem_g[b])

        start_gather(0)
        for k in range(nchunks):
            b = k % _NBUF
            gathers[k].wait()
            outs[k] = pltpu.async_copy(
                rows[b], out_hbm.at[pl.ds(row0 + k * _CHUNK, _CHUNK)],
                sem_o[b])
            if k + 1 < nchunks:
                if k + 1 - _NBUF >= 0:
                    outs[k + 1 - _NBUF].wait()
                start_gather(k + 1)
        for k in range(max(0, nchunks - _NBUF), nchunks):
            outs[k].wait()

    @pl.when(is_hi)
    def _():
        run(_K_HI)

    @pl.when(jnp.logical_not(is_hi))
    def _():
        run(_K_LO)

    @pl.when(wid == _NW - 1)
    def _():
        t0 = _FULL * _CHUNK
        pltpu.async_copy(idx_hbm.at[pl.ds(t0, _TAIL)], idx_t, sem_t).wait()
        pltpu.async_copy(table_hbm.at[idx_t], rows_t, sem_t).wait()
        pltpu.async_copy(rows_t, out_hbm.at[pl.ds(t0, _TAIL)], sem_t).wait()


def kernel(node_specie, embeddings, atomic_masses):
    scaled = pl.pallas_call(
        _prescale_body,
        out_shape=jax.ShapeDtypeStruct((_NUM_SPECIES, _DIM), jnp.float32),
    )(embeddings, atomic_masses[:_NUM_SPECIES].reshape(_NUM_SPECIES, 1))
    return _sc_gather(scaled, node_specie)
